# baseline (speedup 1.0000x reference)
import functools

import jax
import jax.numpy as jnp
from jax import lax
from jax.experimental import pallas as pl
from jax.experimental.pallas import tpu as pltpu
from jax.experimental.pallas import tpu_sc as plsc

LC, L1, L2, D, B, V = 1024, 512, 512, 128, 256, 21
LTOT = LC + L1 + L2
TS = V
QL = LTOT // 4
GS = 32


def _tc_logprob_body(cx_ref, b1_ref, b2_ref, w_ref, wt_ref, out_ref):
    x_all = jnp.concatenate([cx_ref[...], b1_ref[...], b2_ref[...]], axis=0)
    logits = jnp.dot(x_all, w_ref[...], preferred_element_type=jnp.float32)
    row = lax.broadcasted_iota(jnp.int32, (LTOT, 1), 0)
    sign = jnp.where(row < LC, -1.0, 1.0)
    col = lax.broadcasted_iota(jnp.int32, (LTOT, V), 1)
    wt_val = jnp.sum(jnp.where(col == wt_ref[...], logits, 0.0), axis=1,
                     keepdims=True)
    out_ref[...] = (logits - wt_val) * sign


@functools.cache
def _make_sc_gather():
    mesh = plsc.VectorSubcoreMesh(core_axis_name="c", subcore_axis_name="s")

    @functools.partial(
        pl.kernel,
        mesh=mesh,
        out_type=jax.ShapeDtypeStruct((4, B), jnp.float32),
        compiler_params=pltpu.CompilerParams(needs_layout_passes=False),
        scratch_types=[
            pltpu.VMEM((GS * QL,), jnp.int32),
            pltpu.VMEM((QL * TS,), jnp.float32),
            pltpu.VMEM((GS,), jnp.float32),
            pltpu.SemaphoreType.DMA,
        ],
    )
    def sc_gather(cm_hbm, b1m_hbm, b2m_hbm, lp_hbm, out_hbm,
                  seq_v, lp_v, out_v, sem):
        wid = lax.axis_index("s") * 2 + lax.axis_index("c")
        q = wid // 8
        g = wid % 8
        row0 = g * GS

        @pl.when(q == 0)
        def _():
            for k in range(GS):
                pltpu.make_async_copy(
                    cm_hbm.at[row0 + k, pl.ds(0, QL)],
                    seq_v.at[pl.ds(k * QL, QL)], sem).start()

        @pl.when(q == 1)
        def _():
            for k in range(GS):
                pltpu.make_async_copy(
                    cm_hbm.at[row0 + k, pl.ds(QL, QL)],
                    seq_v.at[pl.ds(k * QL, QL)], sem).start()

        @pl.when(q == 2)
        def _():
            for k in range(GS):
                pltpu.make_async_copy(
                    b1m_hbm.at[row0 + k, :],
                    seq_v.at[pl.ds(k * QL, QL)], sem).start()

        @pl.when(q == 3)
        def _():
            for k in range(GS):
                pltpu.make_async_copy(
                    b2m_hbm.at[row0 + k, :],
                    seq_v.at[pl.ds(k * QL, QL)], sem).start()

        with jax.named_scope("table_dma"):
            pltpu.sync_copy(lp_hbm.at[pl.ds(q * QL * TS, QL * TS)], lp_v)
        with jax.named_scope("seq_drain"):
            for k in range(GS):
                pltpu.make_async_copy(
                    cm_hbm.at[row0 + k, pl.ds(0, QL)],
                    seq_v.at[pl.ds(k * QL, QL)], sem).wait()

        lane = lax.broadcasted_iota(jnp.int32, (16,), 0)
        pos_a = lane * QL
        pos_b = pos_a + 16 * QL

        def body(l, accs):
            acc_a, acc_b = accs
            sva = plsc.load_gather(seq_v, [pos_a + l])
            svb = plsc.load_gather(seq_v, [pos_b + l])
            tbase = l * TS
            ta = plsc.load_gather(lp_v, [sva + tbase])
            tb = plsc.load_gather(lp_v, [svb + tbase])
            return acc_a + ta, acc_b + tb

        zero = jnp.zeros((16,), jnp.float32)
        with jax.named_scope("gather_loop"):
            acc_a, acc_b = lax.fori_loop(0, QL, body, (zero, zero), unroll=8)
        out_v[pl.ds(0, 16)] = acc_a
        out_v[pl.ds(16, 16)] = acc_b
        pltpu.sync_copy(out_v, out_hbm.at[q, pl.ds(row0, GS)])

    return sc_gather


def kernel(complex_x, binder1_x, binder2_x, W,
           complex_mut_seqs, binder1_mut_seqs, binder2_mut_seqs,
           complex_wt_seq, binder1_wt_seq, binder2_wt_seq):
    wt_all = jnp.concatenate(
        [complex_wt_seq, binder1_wt_seq, binder2_wt_seq]
    ).astype(jnp.int32).reshape(LTOT, 1)

    lp_adj = pl.pallas_call(
        _tc_logprob_body,
        out_shape=jax.ShapeDtypeStruct((LTOT, V), jnp.float32),
    )(complex_x, binder1_x, binder2_x, W, wt_all)

    partials = _make_sc_gather()(
        complex_mut_seqs.astype(jnp.int32),
        binder1_mut_seqs.astype(jnp.int32),
        binder2_mut_seqs.astype(jnp.int32),
        lp_adj.reshape(LTOT * TS),
    )
    return (partials[0] + partials[1]) + (partials[2] + partials[3])


# speedup vs baseline: 1.2378x; 1.2378x over previous
import functools

import jax
import jax.numpy as jnp
from jax import lax
from jax.experimental import pallas as pl
from jax.experimental.pallas import tpu as pltpu
from jax.experimental.pallas import tpu_sc as plsc

LC, L1, L2, D, B, V = 1024, 512, 512, 128, 256, 21
LTOT = LC + L1 + L2
TS = V
QL = LTOT // 4
GS = 32


def _tc_logprob_body(cx_ref, b1_ref, b2_ref, w_ref, wt_ref, out_ref):
    x_all = jnp.concatenate([cx_ref[...], b1_ref[...], b2_ref[...]], axis=0)
    logits = jnp.dot(x_all, w_ref[...], preferred_element_type=jnp.float32)
    row = lax.broadcasted_iota(jnp.int32, (LTOT, 1), 0)
    sign = jnp.where(row < LC, -1.0, 1.0)
    col = lax.broadcasted_iota(jnp.int32, (LTOT, V), 1)
    wt_val = jnp.sum(jnp.where(col == wt_ref[...], logits, 0.0), axis=1,
                     keepdims=True)
    out_ref[...] = (logits - wt_val) * sign


@functools.cache
def _make_sc_gather():
    mesh = plsc.VectorSubcoreMesh(core_axis_name="c", subcore_axis_name="s")

    @functools.partial(
        pl.kernel,
        mesh=mesh,
        out_type=jax.ShapeDtypeStruct((4, B), jnp.float32),
        compiler_params=pltpu.CompilerParams(needs_layout_passes=False),
        scratch_types=[
            pltpu.VMEM((GS * QL,), jnp.int32),
            pltpu.VMEM((QL * TS,), jnp.float32),
            pltpu.VMEM((GS,), jnp.float32),
            pltpu.SemaphoreType.DMA,
        ],
    )
    def sc_gather(cm_hbm, b1m_hbm, b2m_hbm, lp_hbm, out_hbm,
                  seq_v, lp_v, out_v, sem):
        wid = lax.axis_index("s") * 2 + lax.axis_index("c")
        q = wid // 8
        g = wid % 8
        row0 = g * GS

        @pl.when(q == 0)
        def _():
            for k in range(GS):
                pltpu.make_async_copy(
                    cm_hbm.at[row0 + k, pl.ds(0, QL)],
                    seq_v.at[pl.ds(k * QL, QL)], sem).start()

        @pl.when(q == 1)
        def _():
            for k in range(GS):
                pltpu.make_async_copy(
                    cm_hbm.at[row0 + k, pl.ds(QL, QL)],
                    seq_v.at[pl.ds(k * QL, QL)], sem).start()

        @pl.when(q == 2)
        def _():
            for k in range(GS):
                pltpu.make_async_copy(
                    b1m_hbm.at[row0 + k, :],
                    seq_v.at[pl.ds(k * QL, QL)], sem).start()

        @pl.when(q == 3)
        def _():
            for k in range(GS):
                pltpu.make_async_copy(
                    b2m_hbm.at[row0 + k, :],
                    seq_v.at[pl.ds(k * QL, QL)], sem).start()

        with jax.named_scope("table_dma"):
            pltpu.sync_copy(lp_hbm.at[pl.ds(q * QL * TS, QL * TS)], lp_v)
        with jax.named_scope("seq_drain"):
            for k in range(GS):
                pltpu.make_async_copy(
                    cm_hbm.at[row0 + k, pl.ds(0, QL)],
                    seq_v.at[pl.ds(k * QL, QL)], sem).wait()

        lane = lax.broadcasted_iota(jnp.int32, (16,), 0)
        row_a = lane * QL
        row_b = row_a + 16 * QL

        def body(l, accs):
            acc_a, acc_b = accs
            pa = (lane + l) & (QL - 1)
            sva = plsc.load_gather(seq_v, [row_a + pa])
            svb = plsc.load_gather(seq_v, [row_b + pa])
            p21 = pa * TS
            ta = plsc.load_gather(lp_v, [p21 + sva])
            tb = plsc.load_gather(lp_v, [p21 + svb])
            return acc_a + ta, acc_b + tb

        zero = jnp.zeros((16,), jnp.float32)
        with jax.named_scope("gather_loop"):
            acc_a, acc_b = lax.fori_loop(0, QL, body, (zero, zero), unroll=8)
        out_v[pl.ds(0, 16)] = acc_a
        out_v[pl.ds(16, 16)] = acc_b
        pltpu.sync_copy(out_v, out_hbm.at[q, pl.ds(row0, GS)])

    return sc_gather


def kernel(complex_x, binder1_x, binder2_x, W,
           complex_mut_seqs, binder1_mut_seqs, binder2_mut_seqs,
           complex_wt_seq, binder1_wt_seq, binder2_wt_seq):
    wt_all = jnp.concatenate(
        [complex_wt_seq, binder1_wt_seq, binder2_wt_seq]
    ).astype(jnp.int32).reshape(LTOT, 1)

    lp_adj = pl.pallas_call(
        _tc_logprob_body,
        out_shape=jax.ShapeDtypeStruct((LTOT, V), jnp.float32),
    )(complex_x, binder1_x, binder2_x, W, wt_all)

    partials = _make_sc_gather()(
        complex_mut_seqs.astype(jnp.int32),
        binder1_mut_seqs.astype(jnp.int32),
        binder2_mut_seqs.astype(jnp.int32),
        lp_adj.reshape(LTOT * TS),
    )
    return (partials[0] + partials[1]) + (partials[2] + partials[3])


# speedup vs baseline: 1.4207x; 1.1477x over previous
import functools

import jax
import jax.numpy as jnp
from jax import lax
from jax.experimental import pallas as pl
from jax.experimental.pallas import tpu as pltpu
from jax.experimental.pallas import tpu_sc as plsc

LC, L1, L2, D, B, V = 1024, 512, 512, 128, 256, 21
LTOT = LC + L1 + L2
QL = LTOT // 4
GS = 32


def _tc_logit_body(cx_ref, b1_ref, b2_ref, wt_ref, out_ref):
    x_all = jnp.concatenate([cx_ref[...], b1_ref[...], b2_ref[...]], axis=0)
    logits_t = lax.dot_general(wt_ref[...], x_all, (((1,), (1,)), ((), ())),
                               preferred_element_type=jnp.float32)
    col = lax.broadcasted_iota(jnp.int32, (1, LTOT), 1)
    sign = jnp.where(col < LC, -1.0, 1.0)
    out_ref[...] = logits_t * sign


@functools.cache
def _make_sc_gather():
    mesh = plsc.VectorSubcoreMesh(core_axis_name="c", subcore_axis_name="s")

    @functools.partial(
        pl.kernel,
        mesh=mesh,
        out_type=jax.ShapeDtypeStruct((4, B), jnp.float32),
        compiler_params=pltpu.CompilerParams(needs_layout_passes=False),
        scratch_types=[
            pltpu.VMEM((GS * QL,), jnp.int32),
            pltpu.VMEM((V * QL,), jnp.float32),
            pltpu.VMEM((QL,), jnp.int32),
            pltpu.VMEM((GS,), jnp.float32),
            pltpu.SemaphoreType.DMA,
        ],
    )
    def sc_gather(cm_hbm, b1m_hbm, b2m_hbm, cwt_hbm, b1wt_hbm, b2wt_hbm,
                  lp_hbm, out_hbm, seq_v, lp_v, wt_v, out_v, sem):
        wid = lax.axis_index("s") * 2 + lax.axis_index("c")
        q = wid // 8
        g = wid % 8
        row0 = g * GS

        @pl.when(q == 0)
        def _():
            for k in range(GS):
                pltpu.make_async_copy(
                    cm_hbm.at[row0 + k, pl.ds(0, QL)],
                    seq_v.at[pl.ds(k * QL, QL)], sem).start()
            pltpu.make_async_copy(cwt_hbm.at[pl.ds(0, QL)], wt_v, sem).start()

        @pl.when(q == 1)
        def _():
            for k in range(GS):
                pltpu.make_async_copy(
                    cm_hbm.at[row0 + k, pl.ds(QL, QL)],
                    seq_v.at[pl.ds(k * QL, QL)], sem).start()
            pltpu.make_async_copy(cwt_hbm.at[pl.ds(QL, QL)], wt_v, sem).start()

        @pl.when(q == 2)
        def _():
            for k in range(GS):
                pltpu.make_async_copy(
                    b1m_hbm.at[row0 + k, :],
                    seq_v.at[pl.ds(k * QL, QL)], sem).start()
            pltpu.make_async_copy(b1wt_hbm.at[pl.ds(0, QL)], wt_v, sem).start()

        @pl.when(q == 3)
        def _():
            for k in range(GS):
                pltpu.make_async_copy(
                    b2m_hbm.at[row0 + k, :],
                    seq_v.at[pl.ds(k * QL, QL)], sem).start()
            pltpu.make_async_copy(b2wt_hbm.at[pl.ds(0, QL)], wt_v, sem).start()

        with jax.named_scope("table_dma"):
            for v in range(V):
                pltpu.make_async_copy(
                    lp_hbm.at[pl.ds(v * LTOT + q * QL, QL)],
                    lp_v.at[pl.ds(v * QL, QL)], sem).start()
            for _ in range(GS + 1 + V):
                pltpu.make_async_copy(
                    cm_hbm.at[row0, pl.ds(0, QL)],
                    seq_v.at[pl.ds(0, QL)], sem).wait()

        lane = lax.broadcasted_iota(jnp.int32, (16,), 0)
        row_a = lane * QL
        row_b = row_a + 16 * QL

        def body(l, accs):
            acc_a, acc_b = accs
            pa = (lane + l) & (QL - 1)
            sva = plsc.load_gather(seq_v, [row_a + pa])
            svb = plsc.load_gather(seq_v, [row_b + pa])
            ta = plsc.load_gather(lp_v, [sva * QL + pa])
            tb = plsc.load_gather(lp_v, [svb * QL + pa])
            return acc_a + ta, acc_b + tb

        zero = jnp.zeros((16,), jnp.float32)
        with jax.named_scope("gather_loop"):
            acc_a, acc_b = lax.fori_loop(0, QL, body, (zero, zero), unroll=8)

        def wt_body(j, acc):
            p = j * 16 + lane
            wv = wt_v[pl.ds(j * 16, 16)]
            return acc + plsc.load_gather(lp_v, [wv * QL + p])

        with jax.named_scope("wt_loop"):
            acc_w = lax.fori_loop(0, QL // 16, wt_body, zero, unroll=4)
        c_q = jnp.sum(acc_w)

        out_v[pl.ds(0, 16)] = acc_a - c_q
        out_v[pl.ds(16, 16)] = acc_b - c_q
        pltpu.sync_copy(out_v, out_hbm.at[q, pl.ds(row0, GS)])

    return sc_gather


def kernel(complex_x, binder1_x, binder2_x, W,
           complex_mut_seqs, binder1_mut_seqs, binder2_mut_seqs,
           complex_wt_seq, binder1_wt_seq, binder2_wt_seq):
    lp_t = pl.pallas_call(
        _tc_logit_body,
        out_shape=jax.ShapeDtypeStruct((V, LTOT), jnp.float32),
    )(complex_x, binder1_x, binder2_x, jnp.swapaxes(W, 0, 1))

    partials = _make_sc_gather()(
        complex_mut_seqs.astype(jnp.int32),
        binder1_mut_seqs.astype(jnp.int32),
        binder2_mut_seqs.astype(jnp.int32),
        complex_wt_seq.astype(jnp.int32),
        binder1_wt_seq.astype(jnp.int32),
        binder2_wt_seq.astype(jnp.int32),
        lp_t.reshape(V * LTOT),
    )
    return (partials[0] + partials[1]) + (partials[2] + partials[3])


# speedup vs baseline: 1.4276x; 1.0049x over previous
import functools

import jax
import jax.numpy as jnp
from jax import lax
from jax.experimental import pallas as pl
from jax.experimental.pallas import tpu as pltpu
from jax.experimental.pallas import tpu_sc as plsc

LC, L1, L2, D, B, V = 1024, 512, 512, 128, 256, 21
LTOT = LC + L1 + L2
QL = LTOT // 4
GS = 32
VP = 24


def _tc_logit_body(cx_ref, b1_ref, b2_ref, wt_ref, out_ref):
    x_all = jnp.concatenate([cx_ref[...], b1_ref[...], b2_ref[...]], axis=0)
    logits_t = lax.dot_general(wt_ref[...], x_all, (((1,), (1,)), ((), ())),
                               preferred_element_type=jnp.float32)
    col = lax.broadcasted_iota(jnp.int32, (1, LTOT), 1)
    sign = jnp.where(col < LC, -1.0, 1.0)
    out_ref[0:V, :] = logits_t * sign


@functools.cache
def _make_sc_gather():
    mesh = plsc.VectorSubcoreMesh(core_axis_name="c", subcore_axis_name="s")

    @functools.partial(
        pl.kernel,
        mesh=mesh,
        out_type=jax.ShapeDtypeStruct((4, B), jnp.float32),
        compiler_params=pltpu.CompilerParams(needs_layout_passes=False),
        scratch_types=[
            pltpu.VMEM((GS * QL,), jnp.int32),
            pltpu.VMEM((V * QL,), jnp.float32),
            pltpu.VMEM((QL,), jnp.int32),
            pltpu.VMEM((GS,), jnp.float32),
            pltpu.SemaphoreType.DMA,
        ],
    )
    def sc_gather(cm_hbm, b1m_hbm, b2m_hbm, cwt_hbm, b1wt_hbm, b2wt_hbm,
                  lp_hbm, out_hbm, seq_v, lp_v, wt_v, out_v, sem):
        wid = lax.axis_index("s") * 2 + lax.axis_index("c")
        q = wid // 8
        g = wid % 8
        row0 = g * GS

        @pl.when(q == 0)
        def _():
            for k in range(GS):
                pltpu.make_async_copy(
                    cm_hbm.at[row0 + k, pl.ds(0, QL)],
                    seq_v.at[pl.ds(k * QL, QL)], sem).start()
            pltpu.make_async_copy(cwt_hbm.at[pl.ds(0, QL)], wt_v, sem).start()

        @pl.when(q == 1)
        def _():
            for k in range(GS):
                pltpu.make_async_copy(
                    cm_hbm.at[row0 + k, pl.ds(QL, QL)],
                    seq_v.at[pl.ds(k * QL, QL)], sem).start()
            pltpu.make_async_copy(cwt_hbm.at[pl.ds(QL, QL)], wt_v, sem).start()

        @pl.when(q == 2)
        def _():
            for k in range(GS):
                pltpu.make_async_copy(
                    b1m_hbm.at[row0 + k, :],
                    seq_v.at[pl.ds(k * QL, QL)], sem).start()
            pltpu.make_async_copy(b1wt_hbm.at[pl.ds(0, QL)], wt_v, sem).start()

        @pl.when(q == 3)
        def _():
            for k in range(GS):
                pltpu.make_async_copy(
                    b2m_hbm.at[row0 + k, :],
                    seq_v.at[pl.ds(k * QL, QL)], sem).start()
            pltpu.make_async_copy(b2wt_hbm.at[pl.ds(0, QL)], wt_v, sem).start()

        with jax.named_scope("table_dma"):
            for v in range(V):
                pltpu.make_async_copy(
                    lp_hbm.at[pl.ds(v * LTOT + q * QL, QL)],
                    lp_v.at[pl.ds(v * QL, QL)], sem).start()
            for _ in range(GS + 1 + V):
                pltpu.make_async_copy(
                    cm_hbm.at[row0, pl.ds(0, QL)],
                    seq_v.at[pl.ds(0, QL)], sem).wait()

        lane = lax.broadcasted_iota(jnp.int32, (16,), 0)
        row_a = lane * QL
        row_b = row_a + 16 * QL

        def body(l, accs):
            acc_a, acc_b = accs
            pa = (lane + l) & (QL - 1)
            sva = plsc.load_gather(seq_v, [row_a + pa])
            svb = plsc.load_gather(seq_v, [row_b + pa])
            ta = plsc.load_gather(lp_v, [sva * QL + pa])
            tb = plsc.load_gather(lp_v, [svb * QL + pa])
            return acc_a + ta, acc_b + tb

        zero = jnp.zeros((16,), jnp.float32)
        with jax.named_scope("gather_loop"):
            acc_a, acc_b = lax.fori_loop(0, QL, body, (zero, zero), unroll=8)

        def wt_body(j, acc):
            p = j * 16 + lane
            wv = wt_v[pl.ds(j * 16, 16)]
            return acc + plsc.load_gather(lp_v, [wv * QL + p])

        with jax.named_scope("wt_loop"):
            acc_w = lax.fori_loop(0, QL // 16, wt_body, zero, unroll=4)
        c_q = jnp.sum(acc_w)

        out_v[pl.ds(0, 16)] = acc_a - c_q
        out_v[pl.ds(16, 16)] = acc_b - c_q
        pltpu.sync_copy(out_v, out_hbm.at[q, pl.ds(row0, GS)])

    return sc_gather


def kernel(complex_x, binder1_x, binder2_x, W,
           complex_mut_seqs, binder1_mut_seqs, binder2_mut_seqs,
           complex_wt_seq, binder1_wt_seq, binder2_wt_seq):
    lp_t = pl.pallas_call(
        _tc_logit_body,
        out_shape=jax.ShapeDtypeStruct((VP, LTOT), jnp.float32),
    )(complex_x, binder1_x, binder2_x, jnp.swapaxes(W, 0, 1))

    partials = _make_sc_gather()(
        complex_mut_seqs.astype(jnp.int32),
        binder1_mut_seqs.astype(jnp.int32),
        binder2_mut_seqs.astype(jnp.int32),
        complex_wt_seq.astype(jnp.int32),
        binder1_wt_seq.astype(jnp.int32),
        binder2_wt_seq.astype(jnp.int32),
        lp_t.reshape(VP * LTOT),
    )
    return (partials[0] + partials[1]) + (partials[2] + partials[3])
